# baseline (speedup 1.0000x reference)
import functools
import jax
import jax.numpy as jnp
from jax import lax
from jax.experimental import pallas as pl
from jax.experimental.pallas import tpu as pltpu, tpu_sc as plsc

_VOCAB = 100000
_SEED = 42
_PRIOR_TOKENS = 5
_K = 4
_TOP_P = 0.9

_B = 32
_VREGS = _VOCAB // 16
_CH = 4000
_NCH = _VOCAB // _CH
_HBINS = 2048
_U = 10
_HW = _U * _HBINS

_LOG2E = 1.4426950408889634
_RND = 12582912.0
_C = (0.6931471805599453, 0.2402265069591007, 0.05550410866482158,
      0.009618129107628477, 0.0013333558146428443, 0.00015403530393381608)


def _make_xi(input_ids):
    B = input_ids.shape[0]
    prior_ids = jnp.sum(input_ids[:, -_PRIOR_TOKENS:], axis=1).astype(jnp.uint32)

    def one(b, pid):
        hk = jax.random.fold_in(jax.random.key(_SEED + 1), b)
        hash_idx = jax.random.randint(hk, (), 0, _K)
        k = jax.random.key(_SEED)
        k = jax.random.fold_in(k, hash_idx)
        k = jax.random.fold_in(k, pid)
        xi = jax.random.uniform(k, (_VOCAB,), dtype=jnp.float32)
        return jnp.maximum(xi, 1e-12)

    return jax.vmap(one)(jnp.arange(B), prior_ids)


def _lnn_body(xi_ref, o_ref):
    o_ref[...] = jnp.log(-jnp.log(xi_ref[...]))


def _lnn(xi):
    return pl.pallas_call(
        _lnn_body, out_shape=jax.ShapeDtypeStruct(xi.shape, jnp.float32)
    )(xi)


def _ukey(v):
    bu = plsc.bitcast(v, jnp.uint32)
    return jnp.where(bu >= jnp.uint32(0x80000000), ~bu,
                     bu | jnp.uint32(0x80000000))


def _expapprox(v):
    y = v * _LOG2E
    t = y + _RND
    r = y - (t - _RND)
    ni = plsc.bitcast(t, jnp.int32) - 0x4B400000
    p = r * _C[5] + _C[4]
    p = r * p + _C[3]
    p = r * p + _C[2]
    p = r * p + _C[1]
    p = r * p + _C[0]
    p = r * p + 1.0
    return plsc.bitcast(plsc.bitcast(p, jnp.int32) + lax.shift_left(ni, 23),
                        jnp.float32)


def _crossing(hist, nvregs, theta, a0):
    iota16 = lax.iota(jnp.int32, 16)

    def body(j, carry):
        a, found, bin_, a_above = carry
        jj = nvregs - 1 - j
        h = hist[pl.ds(jj * 16, 16)]
        rev = lax.rev(h, (0,))
        c = plsc.cumsum(rev)
        tot = jnp.sum(h)
        cross = jnp.logical_and(jnp.logical_not(found), (a + tot) >= theta)
        cm = (a + c) >= theta
        ffs = plsc.all_reduce_ffs(cm)
        val_c = jnp.sum(jnp.where(iota16 == ffs, c, 0.0))
        val_r = jnp.sum(jnp.where(iota16 == ffs, rev, 0.0))
        lane = jnp.max(ffs)
        bin_here = jj * 16 + (15 - lane)
        bin_n = jnp.where(cross, bin_here, bin_)
        a_above_n = jnp.where(cross, a + val_c - val_r, a_above)
        return (a + tot, jnp.logical_or(found, cross), bin_n, a_above_n)

    _, _, b, a_above = lax.fori_loop(
        0, nvregs, body, (a0, False, jnp.int32(0), jnp.float32(0.0))
    )
    return b, a_above


def _sc_body(logits_hbm, lnn_hbm, out_hbm, ev, nl0, nl1, hist, sem0, sem1):
    c = lax.axis_index("c")
    s = lax.axis_index("s")
    wid = s * 2 + c
    row = wid * _VOCAB
    iota16 = lax.iota(jnp.int32, 16)

    cp_in = pltpu.make_async_copy(logits_hbm.at[pl.ds(row, _VOCAB)], ev, sem0)
    cp_in.start()
    pltpu.make_async_copy(lnn_hbm.at[pl.ds(row, _CH)], nl0, sem1).start()
    cp_in.wait()

    def zero_hist():
        @plsc.parallel_loop(0, _HW // 16, unroll=8)
        def _(i):
            hist[pl.ds(i * 16, 16)] = jnp.zeros((16,), jnp.float32)

    def merge_banks(nbins):
        @plsc.parallel_loop(0, nbins // 16, unroll=4)
        def _(j):
            m = hist[pl.ds(j * 16, 16)]
            for u in range(1, _U):
                m = m + hist[pl.ds(u * _HBINS + j * 16, 16)]
            hist[pl.ds(j * 16, 16)] = m

    zero_hist()

    def p1(i, _):
        for u in range(_U):
            base = (i * _U + u) * 16
            v = ev[pl.ds(base, 16)]
            e = _expapprox(v)
            k = _ukey(v)
            b = jnp.bitwise_or(
                lax.shift_right_logical(k, jnp.uint32(21)).astype(jnp.int32), u << 11)
            plsc.addupdate_scatter(hist, [b], e)
        return 0

    lax.fori_loop(0, _VREGS // _U, p1, 0)
    merge_banks(_HBINS)

    @plsc.parallel_loop(0, _HBINS // 16, unroll=8,
                        carry=jnp.zeros((16,), jnp.float32))
    def acc(i, a):
        return a + hist[pl.ds(i * 16, 16)]

    theta = _TOP_P * jnp.sum(acc)
    b1, a1 = _crossing(hist, _HBINS // 16, theta, jnp.float32(0.0))
    b1u = b1.astype(jnp.uint32)

    zero_hist()

    def p2(i, _):
        for u in range(_U):
            base = (i * _U + u) * 16
            v = ev[pl.ds(base, 16)]
            e = _expapprox(v)
            k = _ukey(v)
            m = lax.shift_right_logical(k, jnp.uint32(21)) == b1u
            b = jnp.bitwise_or(
                jnp.bitwise_and(lax.shift_right_logical(k, jnp.uint32(10)),
                                jnp.uint32(0x7FF)).astype(jnp.int32),
                u << 11)
            plsc.addupdate_scatter(hist, [b], e, mask=m)
        return 0

    lax.fori_loop(0, _VREGS // _U, p2, 0)
    merge_banks(_HBINS)
    b2, a2 = _crossing(hist, _HBINS // 16, theta, a1)

    pfx = jnp.bitwise_or(lax.shift_left(b1, 11), b2).astype(jnp.uint32)
    zero_hist()

    def p3(i, _):
        for u in range(_U):
            base = (i * _U + u) * 16
            v = ev[pl.ds(base, 16)]
            e = _expapprox(v)
            k = _ukey(v)
            m = lax.shift_right_logical(k, jnp.uint32(10)) == pfx
            b = jnp.bitwise_or(
                jnp.bitwise_and(k, jnp.uint32(0x3FF)).astype(jnp.int32),
                u << 11)
            plsc.addupdate_scatter(hist, [b], e, mask=m)
        return 0

    lax.fori_loop(0, _VREGS // _U, p3, 0)
    merge_banks(1024)
    b3, _ = _crossing(hist, 64, theta, a2)
    tau = jnp.bitwise_or(lax.shift_left(pfx, jnp.uint32(10)), b3.astype(jnp.uint32))

    inf = jnp.full((16,), jnp.inf, jnp.float32)
    neg = jnp.full((16,), -100000.0, jnp.float32)

    carry = (inf, jnp.zeros((16,), jnp.int32))
    for k in range(_NCH):
        nl = nl0 if k % 2 == 0 else nl1
        nxt = nl1 if k % 2 == 0 else nl0
        pltpu.make_async_copy(
            lnn_hbm.at[pl.ds(row + k * _CH, _CH)], nl, sem1
        ).wait()
        if k + 1 < _NCH:
            pltpu.make_async_copy(
                lnn_hbm.at[pl.ds(row + (k + 1) * _CH, _CH)], nxt, sem1
            ).start()
        base = k * _CH

        @plsc.parallel_loop(0, _CH // 16, unroll=5, carry=carry)
        def carry(i, cr):
            best, bidx = cr
            v = ev[pl.ds(base + i * 16, 16)]
            ev[pl.ds(base + i * 16, 16)] = neg
            n = nl[pl.ds(i * 16, 16)]
            sc = jnp.where(_ukey(v) >= tau, n - v, inf)
            upd = sc < best
            gidx = base + i * 16 + iota16
            return jnp.where(upd, sc, best), jnp.where(upd, gidx, bidx)

    best, bidx = carry
    bm = jnp.min(best)
    win = jnp.min(jnp.where(best == bm, bidx, _VOCAB))

    wv = jnp.where(iota16 == (win % 16), 100000.0, -100000.0)
    ev[pl.ds((win // 16) * 16, 16)] = wv
    pltpu.sync_copy(ev, out_hbm.at[pl.ds(row, _VOCAB)])


_sc_call = functools.partial(
    pl.kernel,
    out_type=jax.ShapeDtypeStruct((_B * _VOCAB,), jnp.float32),
    mesh=plsc.VectorSubcoreMesh(core_axis_name="c", subcore_axis_name="s"),
    scratch_types=[
        pltpu.VMEM((_VOCAB,), jnp.float32),
        pltpu.VMEM((_CH,), jnp.float32),
        pltpu.VMEM((_CH,), jnp.float32),
        pltpu.VMEM((_HW,), jnp.float32),
        pltpu.SemaphoreType.DMA,
        pltpu.SemaphoreType.DMA,
    ],
    compiler_params=pltpu.CompilerParams(needs_layout_passes=False),
)(_sc_body)


def kernel(input_ids, logits):
    xi = _make_xi(input_ids)
    lnn = _lnn(xi)
    out = _sc_call(logits.reshape(-1), lnn.reshape(-1))
    return out.reshape(logits.shape)


# speedup vs baseline: 1.6596x; 1.6596x over previous
import functools
import jax
import jax.numpy as jnp
from jax import lax
from jax.experimental import pallas as pl
from jax.experimental.pallas import tpu as pltpu, tpu_sc as plsc

_VOCAB = 100000
_SEED = 42
_PRIOR_TOKENS = 5
_K = 4
_TOP_P = 0.9

_B = 32
_VREGS = _VOCAB // 16
_CH = 4000
_NCH = _VOCAB // _CH
_HBINS = 2048
_U = 10
_HW = _U * _HBINS


def _make_xi(input_ids):
    B = input_ids.shape[0]
    prior_ids = jnp.sum(input_ids[:, -_PRIOR_TOKENS:], axis=1).astype(jnp.uint32)

    def one(b, pid):
        hk = jax.random.fold_in(jax.random.key(_SEED + 1), b)
        hash_idx = jax.random.randint(hk, (), 0, _K)
        k = jax.random.key(_SEED)
        k = jax.random.fold_in(k, hash_idx)
        k = jax.random.fold_in(k, pid)
        xi = jax.random.uniform(k, (_VOCAB,), dtype=jnp.float32)
        return jnp.maximum(xi, 1e-12)

    return jax.vmap(one)(jnp.arange(B), prior_ids)


def _nlx_body(xi_ref, o_ref):
    o_ref[...] = -jnp.log(xi_ref[...])


def _neg_log(xi):
    return pl.pallas_call(
        _nlx_body, out_shape=jax.ShapeDtypeStruct(xi.shape, jnp.float32)
    )(xi)


def _crossing(hist, nvregs, theta, a0):
    iota16 = lax.iota(jnp.int32, 16)

    def body(j, carry):
        a, found, bin_, a_above = carry
        jj = nvregs - 1 - j
        h = hist[pl.ds(jj * 16, 16)]
        rev = lax.rev(h, (0,))
        c = plsc.cumsum(rev)
        tot = jnp.sum(h)
        cross = jnp.logical_and(jnp.logical_not(found), (a + tot) >= theta)
        cm = (a + c) >= theta
        ffs = plsc.all_reduce_ffs(cm)
        val_c = jnp.sum(jnp.where(iota16 == ffs, c, 0.0))
        val_r = jnp.sum(jnp.where(iota16 == ffs, rev, 0.0))
        lane = jnp.max(ffs)
        bin_here = jj * 16 + (15 - lane)
        bin_n = jnp.where(cross, bin_here, bin_)
        a_above_n = jnp.where(cross, a + val_c - val_r, a_above)
        return (a + tot, jnp.logical_or(found, cross), bin_n, a_above_n)

    _, _, b, a_above = lax.fori_loop(
        0, nvregs, body, (a0, False, jnp.int32(0), jnp.float32(0.0))
    )
    return b, a_above


def _sc_body(logits_hbm, nlx_hbm, out_hbm, ev, nl0, nl1, hist, sem0, sem1):
    c = lax.axis_index("c")
    s = lax.axis_index("s")
    wid = s * 2 + c
    row = wid * _VOCAB
    iota16 = lax.iota(jnp.int32, 16)

    cp_in = pltpu.make_async_copy(logits_hbm.at[pl.ds(row, _VOCAB)], ev, sem0)
    cp_in.start()
    pltpu.make_async_copy(nlx_hbm.at[pl.ds(row, _CH)], nl0, sem1).start()
    cp_in.wait()

    def zero_hist():
        @plsc.parallel_loop(0, _HW // 16, unroll=8)
        def _(i):
            hist[pl.ds(i * 16, 16)] = jnp.zeros((16,), jnp.float32)

    def merge_banks(nbins):
        @plsc.parallel_loop(0, nbins // 16, unroll=4)
        def _(j):
            m = hist[pl.ds(j * 16, 16)]
            for u in range(1, _U):
                m = m + hist[pl.ds(u * _HBINS + j * 16, 16)]
            hist[pl.ds(j * 16, 16)] = m

    zero_hist()

    def p1(i, _):
        for u in range(_U):
            base = (i * _U + u) * 16
            v = jnp.exp(ev[pl.ds(base, 16)])
            ev[pl.ds(base, 16)] = v
            bits = plsc.bitcast(v, jnp.int32)
            b = jnp.bitwise_or(lax.shift_right_logical(bits, 21), u << 11)
            plsc.addupdate_scatter(hist, [b], v)
        return 0

    lax.fori_loop(0, _VREGS // _U, p1, 0)
    merge_banks(_HBINS)

    @plsc.parallel_loop(0, _HBINS // 16, unroll=8,
                        carry=jnp.zeros((16,), jnp.float32))
    def acc(i, a):
        return a + hist[pl.ds(i * 16, 16)]

    theta = _TOP_P * jnp.sum(acc)
    b1, a1 = _crossing(hist, _HBINS // 16, theta, jnp.float32(0.0))

    zero_hist()

    def p2(i, _):
        for u in range(_U):
            base = (i * _U + u) * 16
            v = ev[pl.ds(base, 16)]
            bits = plsc.bitcast(v, jnp.int32)
            m = lax.shift_right_logical(bits, 21) == b1
            b = jnp.bitwise_or(
                jnp.bitwise_and(lax.shift_right_logical(bits, 10), 0x7FF),
                u << 11)
            plsc.addupdate_scatter(hist, [b], v, mask=m)
        return 0

    lax.fori_loop(0, _VREGS // _U, p2, 0)
    merge_banks(_HBINS)
    b2, a2 = _crossing(hist, _HBINS // 16, theta, a1)

    pfx = jnp.bitwise_or(lax.shift_left(b1, 11), b2)
    zero_hist()

    def p3(i, _):
        for u in range(_U):
            base = (i * _U + u) * 16
            v = ev[pl.ds(base, 16)]
            bits = plsc.bitcast(v, jnp.int32)
            m = lax.shift_right_logical(bits, 10) == pfx
            b = jnp.bitwise_or(jnp.bitwise_and(bits, 0x3FF), u << 11)
            plsc.addupdate_scatter(hist, [b], v, mask=m)
        return 0

    lax.fori_loop(0, _VREGS // _U, p3, 0)
    merge_banks(1024)
    b3, _ = _crossing(hist, 64, theta, a2)
    tau = jnp.bitwise_or(lax.shift_left(pfx, 10), b3)

    inf = jnp.full((16,), jnp.inf, jnp.float32)
    neg = jnp.full((16,), -100000.0, jnp.float32)

    carry = (inf, jnp.zeros((16,), jnp.int32))
    for k in range(_NCH):
        nl = nl0 if k % 2 == 0 else nl1
        nxt = nl1 if k % 2 == 0 else nl0
        pltpu.make_async_copy(
            nlx_hbm.at[pl.ds(row + k * _CH, _CH)], nl, sem1
        ).wait()
        if k + 1 < _NCH:
            pltpu.make_async_copy(
                nlx_hbm.at[pl.ds(row + (k + 1) * _CH, _CH)], nxt, sem1
            ).start()
        base = k * _CH

        @plsc.parallel_loop(0, _CH // 16, unroll=5, carry=carry)
        def carry(i, cr):
            best, bidx = cr
            v = ev[pl.ds(base + i * 16, 16)]
            ev[pl.ds(base + i * 16, 16)] = neg
            n = nl[pl.ds(i * 16, 16)]
            bits = plsc.bitcast(v, jnp.int32)
            sc = jnp.where(bits >= tau, n / v, inf)
            upd = sc < best
            gidx = base + i * 16 + iota16
            return jnp.where(upd, sc, best), jnp.where(upd, gidx, bidx)

    best, bidx = carry
    bm = jnp.min(best)
    win = jnp.min(jnp.where(best == bm, bidx, _VOCAB))

    wv = jnp.where(iota16 == (win % 16), 100000.0, -100000.0)
    ev[pl.ds((win // 16) * 16, 16)] = wv
    pltpu.sync_copy(ev, out_hbm.at[pl.ds(row, _VOCAB)])


_sc_call = functools.partial(
    pl.kernel,
    out_type=jax.ShapeDtypeStruct((_B * _VOCAB,), jnp.float32),
    mesh=plsc.VectorSubcoreMesh(core_axis_name="c", subcore_axis_name="s"),
    scratch_types=[
        pltpu.VMEM((_VOCAB,), jnp.float32),
        pltpu.VMEM((_CH,), jnp.float32),
        pltpu.VMEM((_CH,), jnp.float32),
        pltpu.VMEM((_HW,), jnp.float32),
        pltpu.SemaphoreType.DMA,
        pltpu.SemaphoreType.DMA,
    ],
    compiler_params=pltpu.CompilerParams(needs_layout_passes=False),
)(_sc_body)


def kernel(input_ids, logits):
    xi = _make_xi(input_ids)
    nlx = _neg_log(xi)
    out = _sc_call(logits.reshape(-1), nlx.reshape(-1))
    return out.reshape(logits.shape)
